# baseline (speedup 1.0000x reference)
import functools

import jax
import jax.numpy as jnp
from jax import lax
from jax.experimental import pallas as pl
from jax.experimental.pallas import tpu as pltpu
from jax.experimental.pallas import tpu_sc as plsc

T, DEPTH, HEIGHT, WIDTH = 3, 72, 512, 512
N = 1048576

NC, NS, L = 2, 16, 16
NW = NC * NS
CHUNK = 128
PER_TILE = N // NW
NCHUNKS = PER_TILE // CHUNK
VPC = CHUNK // L

DY = WIDTH
DZ = HEIGHT * WIDTH
DT = DEPTH * HEIGHT * WIDTH


def _lerp(a, b, w):
  return a + w * (b - a)


def _body(crs, fr, out, cbuf, wbuf, ibuf, vbuf, obuf, sem0, sem1):
  sems = (sem0, sem1)
  wid = lax.axis_index("s") * NC + lax.axis_index("c")
  chunk0 = wid * NCHUNKS

  def fire(j, b):
    pltpu.sync_copy(crs.at[chunk0 + j], cbuf.at[b])
    for v in range(VPC):
      s = pl.ds(v * L, L)
      z = cbuf[b, 0, s]
      y = cbuf[b, 1, s]
      x = cbuf[b, 2, s]
      t = cbuf[b, 3, s]
      sz = z * float(DEPTH - 1)
      sy = y * float(HEIGHT - 1)
      sx = x * float(WIDTH - 1)
      ta = t * float(T)
      iz = sz.astype(jnp.int32)
      iy = sy.astype(jnp.int32)
      ix = sx.astype(jnp.int32)
      it = ta.astype(jnp.int32)
      wbuf[b, 0, s] = sz - iz.astype(jnp.float32)
      wbuf[b, 1, s] = sy - iy.astype(jnp.float32)
      wbuf[b, 2, s] = sx - ix.astype(jnp.float32)
      z0 = jnp.clip(iz, 0, DEPTH - 1)
      y0 = jnp.clip(iy, 0, HEIGHT - 1)
      x0 = jnp.clip(ix, 0, WIDTH - 1)
      t0 = jnp.clip(it, 0, T - 1)
      wbuf[b, 3, s] = ta - t0.astype(jnp.float32)
      z1 = jnp.minimum(z0 + 1, DEPTH - 1)
      y1 = jnp.minimum(y0 + 1, HEIGHT - 1)
      x1 = jnp.minimum(x0 + 1, WIDTH - 1)
      t1 = jnp.minimum(t0 + 1, T - 1)
      tb = (t0 * DT, t1 * DT)
      zb = (z0 * DZ, z1 * DZ)
      yb = (y0 * DY, y1 * DY)
      xs = (x0, x1)
      for zi in range(2):
        for yi in range(2):
          sp = zb[zi] + yb[yi]
          for ti in range(2):
            base = tb[ti] + sp
            for xi in range(2):
              ibuf[b, ti * 8 + zi * 4 + yi * 2 + xi, s] = base + xs[xi]
    for c in range(16):
      pltpu.async_copy(fr.at[ibuf.at[b, c]], vbuf.at[b, c], sems[b])

  def combine(j, b):
    for c in range(16):
      pltpu.make_async_copy(fr.at[ibuf.at[b, c]], vbuf.at[b, c], sems[b]).wait()
    for v in range(VPC):
      s = pl.ds(v * L, L)
      fz = wbuf[b, 0, s]
      fy = wbuf[b, 1, s]
      fx = wbuf[b, 2, s]
      ft = wbuf[b, 3, s]
      vals = []
      for ti in range(2):
        c00 = _lerp(vbuf[b, ti * 8 + 0, s], vbuf[b, ti * 8 + 1, s], fx)
        c01 = _lerp(vbuf[b, ti * 8 + 2, s], vbuf[b, ti * 8 + 3, s], fx)
        c10 = _lerp(vbuf[b, ti * 8 + 4, s], vbuf[b, ti * 8 + 5, s], fx)
        c11 = _lerp(vbuf[b, ti * 8 + 6, s], vbuf[b, ti * 8 + 7, s], fx)
        c0 = _lerp(c00, c01, fy)
        c1 = _lerp(c10, c11, fy)
        vals.append(_lerp(c0, c1, fz))
      obuf[pl.ds(j * CHUNK + v * L, L)] = _lerp(vals[0], vals[1], ft)

  fire(0, 0)

  def loop_body(i, carry):
    j = 2 * i
    combine(j, 0)
    fire(j + 1, 1)
    combine(j + 1, 1)
    fire(j + 2, 0)
    return carry

  lax.fori_loop(0, NCHUNKS // 2 - 1, loop_body, 0)
  combine(NCHUNKS - 2, 0)
  fire(NCHUNKS - 1, 1)
  combine(NCHUNKS - 1, 1)
  pltpu.sync_copy(obuf, out.at[pl.ds(wid * PER_TILE, PER_TILE)])


@jax.jit
def kernel(coords, frames):
  crs = coords.reshape(N // CHUNK, CHUNK, 4).transpose(0, 2, 1)
  fr = frames.reshape(-1)
  mesh = plsc.VectorSubcoreMesh(core_axis_name="c", subcore_axis_name="s")
  out = pl.kernel(
      _body,
      out_type=jax.ShapeDtypeStruct((N,), jnp.float32),
      mesh=mesh,
      scratch_types=[
          pltpu.VMEM((2, 4, CHUNK), jnp.float32),
          pltpu.VMEM((2, 4, CHUNK), jnp.float32),
          pltpu.VMEM((2, 16, CHUNK), jnp.int32),
          pltpu.VMEM((2, 16, CHUNK), jnp.float32),
          pltpu.VMEM((PER_TILE,), jnp.float32),
          pltpu.SemaphoreType.DMA,
          pltpu.SemaphoreType.DMA,
      ],
  )(crs, fr)
  return out[:, None]


# speedup vs baseline: 1.4793x; 1.4793x over previous
import functools

import jax
import jax.numpy as jnp
from jax import lax
from jax.experimental import pallas as pl
from jax.experimental.pallas import tpu as pltpu
from jax.experimental.pallas import tpu_sc as plsc

T, DEPTH, HEIGHT, WIDTH = 3, 72, 512, 512
N = 1048576

NC, NS, L = 2, 16, 16
NW = NC * NS
CHUNK = 128
PER_TILE = N // NW
NCHUNKS = PER_TILE // CHUNK
VPC = CHUNK // L

DY = WIDTH
DZ = HEIGHT * WIDTH
DT = DEPTH * HEIGHT * WIDTH


def _lerp(a, b, w):
  return a + w * (b - a)


def _body(crs, fr, out, cbuf0, cbuf1, wbuf, ibuf, vbuf, obuf, sem0, sem1, csem0, csem1):
  sems = (sem0, sem1)
  csems = (csem0, csem1)
  cbufs = (cbuf0, cbuf1)
  wid = lax.axis_index("s") * NC + lax.axis_index("c")
  chunk0 = wid * NCHUNKS

  def start_coords(j, b):
    jj = chunk0 + jnp.minimum(j, NCHUNKS - 1)
    src = crs.at[:, pl.ds(jj * CHUNK, CHUNK)]
    pltpu.async_copy(src, cbufs[b], csems[b])

  def wait_coords(b):
    pltpu.make_async_copy(crs.at[:, pl.ds(0, CHUNK)], cbufs[b], csems[b]).wait()

  def fire(j, b):
    wait_coords(b)
    start_coords(j + 1, 1 - b)
    cb = cbufs[b]
    for v in range(VPC):
      s = pl.ds(v * L, L)
      z = cb[0, s]
      y = cb[1, s]
      x = cb[2, s]
      t = cb[3, s]
      sz = z * float(DEPTH - 1)
      sy = y * float(HEIGHT - 1)
      sx = x * float(WIDTH - 1)
      ta = t * float(T)
      iz = sz.astype(jnp.int32)
      iy = sy.astype(jnp.int32)
      ix = sx.astype(jnp.int32)
      it = ta.astype(jnp.int32)
      wbuf[b, 0, s] = sz - iz.astype(jnp.float32)
      wbuf[b, 1, s] = sy - iy.astype(jnp.float32)
      wbuf[b, 2, s] = sx - ix.astype(jnp.float32)
      z0 = jnp.clip(iz, 0, DEPTH - 1)
      y0 = jnp.clip(iy, 0, HEIGHT - 1)
      x0 = jnp.clip(ix, 0, WIDTH - 1)
      t0 = jnp.clip(it, 0, T - 1)
      wbuf[b, 3, s] = ta - t0.astype(jnp.float32)
      z1 = jnp.minimum(z0 + 1, DEPTH - 1)
      y1 = jnp.minimum(y0 + 1, HEIGHT - 1)
      x1 = jnp.minimum(x0 + 1, WIDTH - 1)
      t1 = jnp.minimum(t0 + 1, T - 1)
      tb = (t0 * DT, t1 * DT)
      zb = (z0 * DZ, z1 * DZ)
      yb = (y0 * DY, y1 * DY)
      xs = (x0, x1)
      for zi in range(2):
        for yi in range(2):
          sp = zb[zi] + yb[yi]
          for ti in range(2):
            base = tb[ti] + sp
            for xi in range(2):
              ibuf[b, ti * 8 + zi * 4 + yi * 2 + xi, s] = base + xs[xi]
    for c in range(16):
      pltpu.async_copy(fr.at[ibuf.at[b, c]], vbuf.at[b, c], sems[b])

  def combine(j, b):
    for c in range(16):
      pltpu.make_async_copy(fr.at[ibuf.at[b, c]], vbuf.at[b, c], sems[b]).wait()
    for v in range(VPC):
      s = pl.ds(v * L, L)
      fz = wbuf[b, 0, s]
      fy = wbuf[b, 1, s]
      fx = wbuf[b, 2, s]
      ft = wbuf[b, 3, s]
      vals = []
      for ti in range(2):
        c00 = _lerp(vbuf[b, ti * 8 + 0, s], vbuf[b, ti * 8 + 1, s], fx)
        c01 = _lerp(vbuf[b, ti * 8 + 2, s], vbuf[b, ti * 8 + 3, s], fx)
        c10 = _lerp(vbuf[b, ti * 8 + 4, s], vbuf[b, ti * 8 + 5, s], fx)
        c11 = _lerp(vbuf[b, ti * 8 + 6, s], vbuf[b, ti * 8 + 7, s], fx)
        c0 = _lerp(c00, c01, fy)
        c1 = _lerp(c10, c11, fy)
        vals.append(_lerp(c0, c1, fz))
      obuf[pl.ds(j * CHUNK + v * L, L)] = _lerp(vals[0], vals[1], ft)

  start_coords(0, 0)
  fire(0, 0)

  def loop_body(i, carry):
    j = 2 * i
    fire(j + 1, 1)
    combine(j, 0)
    fire(j + 2, 0)
    combine(j + 1, 1)
    return carry

  lax.fori_loop(0, NCHUNKS // 2 - 1, loop_body, 0)
  fire(NCHUNKS - 1, 1)
  combine(NCHUNKS - 2, 0)
  combine(NCHUNKS - 1, 1)
  wait_coords(0)
  pltpu.sync_copy(obuf, out.at[pl.ds(wid * PER_TILE, PER_TILE)])


@jax.jit
def kernel(coords, frames):
  crs = coords.T
  fr = frames.reshape(-1)
  mesh = plsc.VectorSubcoreMesh(core_axis_name="c", subcore_axis_name="s")
  out = pl.kernel(
      _body,
      out_type=jax.ShapeDtypeStruct((N,), jnp.float32),
      mesh=mesh,
      scratch_types=[
          pltpu.VMEM((4, CHUNK), jnp.float32),
          pltpu.VMEM((4, CHUNK), jnp.float32),
          pltpu.VMEM((2, 4, CHUNK), jnp.float32),
          pltpu.VMEM((2, 16, CHUNK), jnp.int32),
          pltpu.VMEM((2, 16, CHUNK), jnp.float32),
          pltpu.VMEM((PER_TILE,), jnp.float32),
          pltpu.SemaphoreType.DMA,
          pltpu.SemaphoreType.DMA,
          pltpu.SemaphoreType.DMA,
          pltpu.SemaphoreType.DMA,
      ],
  )(crs, fr)
  return out[:, None]
